# baseline (speedup 1.0000x reference)
import functools

import jax
import jax.numpy as jnp
from jax import lax
from jax.experimental import pallas as pl
from jax.experimental.pallas import tpu as pltpu
from jax.experimental.pallas import tpu_sc as plsc

_D = 256
_B = 100000
_NC = 2
_NS = 16
_NW = _NC * _NS
_C = 128
_FULL = 24
_TAIL = 56
_BPW = _FULL * _C + _TAIL
_IPW = (_FULL + 1) * _C
_BPAD = _BPW * _NW

_mesh = plsc.VectorSubcoreMesh(core_axis_name="c", subcore_axis_name="s")


@functools.partial(
    pl.kernel,
    out_type=jax.ShapeDtypeStruct((_BPAD, _D), jnp.float32),
    mesh=_mesh,
    scratch_types=[
        pltpu.VMEM((_IPW,), jnp.int32),
        pltpu.VMEM((2, _C, _D), jnp.float32),
        pltpu.SemaphoreType.DMA,
        pltpu.SemaphoreType.DMA,
    ],
)
def _gather_rows(x_hbm, idx_hbm, out_hbm, idx_v, rows_v, sem0, sem1):
    wid = lax.axis_index("s") * _NC + lax.axis_index("c")
    base = wid * _BPW

    pltpu.sync_copy(idx_hbm.at[pl.ds(wid * _IPW, _IPW)], idx_v)

    sems = (sem0, sem1)

    def start_gather(slot, chunk):
        pltpu.async_copy(
            x_hbm.at[idx_v.at[pl.ds(chunk * _C, _C)]], rows_v.at[slot], sems[slot]
        )

    def wait_gather(slot):
        pltpu.make_async_copy(
            x_hbm.at[idx_v.at[pl.ds(0, _C)]], rows_v.at[slot], sems[slot]
        ).wait()

    start_gather(0, 0)
    start_gather(1, 1)

    @pl.loop(0, _FULL, step=2)
    def _(j):
        for b in range(2):
            cj = j + b
            wait_gather(b)
            pltpu.sync_copy(
                rows_v.at[b], out_hbm.at[pl.ds(base + cj * _C, _C)]
            )

            @pl.when(cj + 2 <= _FULL)
            def _():
                start_gather(b, cj + 2)

    wait_gather(0)
    pltpu.sync_copy(
        rows_v.at[0, pl.ds(0, _TAIL)],
        out_hbm.at[pl.ds(base + _FULL * _C, _TAIL)],
    )


def kernel(x, inds):
    idx = inds[:, 0].astype(jnp.int32)
    idx = jnp.concatenate([idx, jnp.zeros((_BPAD - _B,), jnp.int32)])
    idx = idx.reshape(_NW, _BPW)
    idx = jnp.pad(idx, ((0, 0), (0, _IPW - _BPW))).reshape(-1)
    out = _gather_rows(x, idx)
    return out[:_B]


# speedup vs baseline: 1.4327x; 1.4327x over previous
import functools

import jax
import jax.numpy as jnp
from jax import lax
from jax.experimental import pallas as pl
from jax.experimental.pallas import tpu as pltpu
from jax.experimental.pallas import tpu_sc as plsc

_D = 256
_B = 100000
_NC = 2
_NS = 16
_NW = _NC * _NS
_C = 112
_NBUF = 4
_STEPS = 28
_TAIL = 104
_BPW = (_STEPS - 1) * _C + _TAIL
_IPW = _STEPS * _C
_BPAD = _BPW * _NW
_LOOPED = _STEPS - 4

_mesh = plsc.VectorSubcoreMesh(core_axis_name="c", subcore_axis_name="s")


@functools.partial(
    pl.kernel,
    out_type=jax.ShapeDtypeStruct((_BPAD, _D), jnp.float32),
    mesh=_mesh,
    scratch_types=[
        pltpu.VMEM((_IPW,), jnp.int32),
        pltpu.VMEM((_NBUF, _C, _D), jnp.float32),
    ]
    + [pltpu.SemaphoreType.DMA] * (2 * _NBUF),
)
def _gather_rows(x_hbm, idx_hbm, out_hbm, idx_v, rows_v, *sems):
    gsems = sems[:_NBUF]
    wsems = sems[_NBUF:]
    wid = lax.axis_index("s") * _NC + lax.axis_index("c")
    base = wid * _BPW

    pltpu.sync_copy(idx_hbm.at[pl.ds(wid * _IPW, _IPW)], idx_v)

    def start_gather(slot, chunk):
        pltpu.async_copy(
            x_hbm.at[idx_v.at[pl.ds(chunk * _C, _C)]], rows_v.at[slot], gsems[slot]
        )

    def wait_gather(slot):
        pltpu.make_async_copy(
            x_hbm.at[idx_v.at[pl.ds(0, _C)]], rows_v.at[slot], gsems[slot]
        ).wait()

    def start_write(slot, chunk, rows=_C):
        pltpu.async_copy(
            rows_v.at[slot, pl.ds(0, rows)],
            out_hbm.at[pl.ds(base + chunk * _C, rows)],
            wsems[slot],
        )

    def wait_write(slot, rows=_C):
        pltpu.make_async_copy(
            rows_v.at[slot, pl.ds(0, rows)],
            out_hbm.at[pl.ds(base, rows)],
            wsems[slot],
        ).wait()

    start_gather(0, 0)
    start_gather(1, 1)

    @pl.loop(0, _LOOPED, step=_NBUF)
    def _(j):
        for b in range(_NBUF):
            cj = j + b
            slot = b
            nslot = (b + 2) % _NBUF
            wait_gather(slot)
            start_write(slot, cj)

            @pl.when(cj >= 2)
            def _():
                wait_write(nslot)

            start_gather(nslot, cj + 2)

    for t in range(_LOOPED, _STEPS):
        slot = t % _NBUF
        wait_gather(slot)
        if t == _STEPS - 1:
            start_write(slot, t, _TAIL)
        else:
            start_write(slot, t)
        if t + 2 < _STEPS:
            nslot = (t + 2) % _NBUF
            wait_write(nslot)
            start_gather(nslot, t + 2)

    for t in range(_STEPS - _NBUF, _STEPS):
        wait_write(t % _NBUF, _TAIL if t == _STEPS - 1 else _C)


def kernel(x, inds):
    idx = inds[:, 0].astype(jnp.int32)
    idx = jnp.concatenate([idx, jnp.zeros((_BPAD - _B,), jnp.int32)])
    idx = idx.reshape(_NW, _BPW)
    idx = jnp.pad(idx, ((0, 0), (0, _IPW - _BPW))).reshape(-1)
    out = _gather_rows(x, idx)
    return out[:_B]
